# Initial kernel scaffold; baseline (speedup 1.0000x reference)
import jax
import jax.numpy as jnp
from jax.experimental import pallas as pl


def kernel(features, rois):
    raise NotImplementedError("write your pallas kernel here")


# speedup vs baseline: 1.2552x; 1.2552x over previous
import functools

import jax
import jax.numpy as jnp
from jax import lax
from jax.experimental import pallas as pl
from jax.experimental.pallas import tpu as pltpu
from jax.experimental.pallas import tpu_sc as plsc

OUT_H = 7
OUT_W = 7
SPATIAL_SCALE = 0.125
N, C, H, W = 2, 256, 128, 128
K = 1000
KP = 1024
G = OUT_H * OUT_W
P = G * KP
CH = 32
LANES = 16


def _tc_index_kernel(rt_ref, idx_ref, w_ref):
    bi = rt_ref[0:1, :].astype(jnp.int32)
    cx = rt_ref[1:2, :]
    cy = rt_ref[2:3, :]
    rw = rt_ref[3:4, :]
    rh = rt_ref[4:5, :]
    ct = rt_ref[5:6, :]
    st = rt_ref[6:7, :]
    gi = lax.broadcasted_iota(jnp.int32, (G, KP), 0)
    gxf = (gi % OUT_W).astype(jnp.float32)
    gyf = (gi // OUT_W).astype(jnp.float32)
    gx = (gxf + 0.5) / OUT_W - 0.5
    gy = (gyf + 0.5) / OUT_H - 0.5
    gxw = gx * rw
    gyh = gy * rh
    ix = gxw * ct - gyh * st + cx - 0.5
    iy = gxw * st + gyh * ct + cy - 0.5
    x0 = jnp.floor(ix)
    y0 = jnp.floor(iy)
    wx1 = ix - x0
    wx0 = 1.0 - wx1
    wy1 = iy - y0
    wy0 = 1.0 - wy1
    bbase = bi * (H * W)
    corners = (
        (x0, y0, wx0 * wy0),
        (x0 + 1.0, y0, wx1 * wy0),
        (x0, y0 + 1.0, wx0 * wy1),
        (x0 + 1.0, y0 + 1.0, wx1 * wy1),
    )
    for j, (xc, yc, wj) in enumerate(corners):
        valid = (xc >= 0.0) & (xc <= W - 1.0) & (yc >= 0.0) & (yc <= H - 1.0)
        xi = jnp.clip(xc, 0.0, W - 1.0).astype(jnp.int32)
        yi = jnp.clip(yc, 0.0, H - 1.0).astype(jnp.int32)
        idx_ref[j] = bbase + yi * W + xi
        w_ref[j] = jnp.where(valid, wj, 0.0)


_tc_index = pl.pallas_call(
    _tc_index_kernel,
    out_shape=(
        jax.ShapeDtypeStruct((4, G, KP), jnp.int32),
        jax.ShapeDtypeStruct((4, G, KP), jnp.float32),
    ),
)


def _make_sc_gather(nw):
    per_w = P // nw
    n_chunks = per_w // CH
    mesh = plsc.VectorSubcoreMesh(core_axis_name="c", subcore_axis_name="s")

    @functools.partial(
        pl.kernel,
        mesh=mesh,
        out_type=jax.ShapeDtypeStruct((P, C), jnp.float32),
        scratch_types=[
            pltpu.VMEM((4 * CH,), jnp.int32),
            pltpu.VMEM((4 * CH, LANES), jnp.float32),
            pltpu.VMEM((4 * CH, C), jnp.float32),
            pltpu.VMEM((CH, C), jnp.float32),
            pltpu.SemaphoreType.DMA,
        ],
    )
    def sc_fn(feats_hbm, idx_hbm, w_hbm, out_hbm, idx_v, w_v, rows_v, out_v, sem):
        wid = lax.axis_index("s") * 2 + lax.axis_index("c")

        def chunk_body(ck, carry):
            gc = wid * n_chunks + ck
            start = wid * per_w + ck * CH
            pltpu.sync_copy(idx_hbm.at[gc], idx_v)
            pltpu.sync_copy(w_hbm.at[gc], w_v)
            pltpu.async_copy(feats_hbm.at[idx_v], rows_v, sem).wait()

            def pbody(p, c2):
                wb0 = w_v[p, :]
                wb1 = w_v[CH + p, :]
                wb2 = w_v[2 * CH + p, :]
                wb3 = w_v[3 * CH + p, :]
                for cc in range(C // LANES):
                    s = pl.ds(cc * LANES, LANES)
                    acc = rows_v[p, s] * wb0
                    acc = acc + rows_v[CH + p, s] * wb1
                    acc = acc + rows_v[2 * CH + p, s] * wb2
                    acc = acc + rows_v[3 * CH + p, s] * wb3
                    out_v[p, s] = acc
                return c2

            lax.fori_loop(0, CH, pbody, 0)
            pltpu.sync_copy(out_v, out_hbm.at[pl.ds(start, CH)])
            return carry

        lax.fori_loop(0, n_chunks, chunk_body, 0)

    return sc_fn


@functools.cache
def _sc_gather_cached():
    return _make_sc_gather(32)


def kernel(features, rois):
    feats_flat = jnp.transpose(features, (0, 2, 3, 1)).reshape(N * H * W, C)
    th = rois[:, 5] * SPATIAL_SCALE
    rt = jnp.stack(
        [
            rois[:, 0],
            rois[:, 1] * SPATIAL_SCALE,
            rois[:, 2] * SPATIAL_SCALE,
            rois[:, 3] * SPATIAL_SCALE,
            rois[:, 4] * SPATIAL_SCALE,
            jnp.cos(th),
            jnp.sin(th),
            jnp.zeros_like(th),
        ],
        axis=0,
    )
    rt = jnp.pad(rt, ((0, 0), (0, KP - K)))
    idx4, w4 = _tc_index(rt)
    idx_sc = idx4.reshape(4, P // CH, CH).transpose(1, 0, 2).reshape(P // CH, 4 * CH)
    w_sc = w4.reshape(4, P // CH, CH).transpose(1, 0, 2).reshape(P // CH, 4 * CH)
    w_sc = jnp.broadcast_to(w_sc[:, :, None], (P // CH, 4 * CH, LANES))
    out2 = _sc_gather_cached()(feats_flat, idx_sc, w_sc)
    out = out2.reshape(G, KP, C)[:, :K]
    return out.transpose(1, 2, 0).reshape(K, C, OUT_H, OUT_W)


# speedup vs baseline: 1.4578x; 1.1614x over previous
import functools

import jax
import jax.numpy as jnp
from jax import lax
from jax.experimental import pallas as pl
from jax.experimental.pallas import tpu as pltpu
from jax.experimental.pallas import tpu_sc as plsc

OUT_H = 7
OUT_W = 7
SPATIAL_SCALE = 0.125
N, C, H, W = 2, 256, 128, 128
K = 1000
KP = 1024
G = OUT_H * OUT_W
P = G * KP
CH = 32
LANES = 16


def _tc_index_kernel(rt_ref, idx_ref, w_ref):
    bi = rt_ref[0:1, :].astype(jnp.int32)
    cx = rt_ref[1:2, :]
    cy = rt_ref[2:3, :]
    rw = rt_ref[3:4, :]
    rh = rt_ref[4:5, :]
    ct = rt_ref[5:6, :]
    st = rt_ref[6:7, :]
    gi = lax.broadcasted_iota(jnp.int32, (G, KP), 0)
    gxf = (gi % OUT_W).astype(jnp.float32)
    gyf = (gi // OUT_W).astype(jnp.float32)
    gx = (gxf + 0.5) / OUT_W - 0.5
    gy = (gyf + 0.5) / OUT_H - 0.5
    gxw = gx * rw
    gyh = gy * rh
    ix = gxw * ct - gyh * st + cx - 0.5
    iy = gxw * st + gyh * ct + cy - 0.5
    x0 = jnp.floor(ix)
    y0 = jnp.floor(iy)
    wx1 = ix - x0
    wx0 = 1.0 - wx1
    wy1 = iy - y0
    wy0 = 1.0 - wy1
    bbase = bi * (H * W)
    corners = (
        (x0, y0, wx0 * wy0),
        (x0 + 1.0, y0, wx1 * wy0),
        (x0, y0 + 1.0, wx0 * wy1),
        (x0 + 1.0, y0 + 1.0, wx1 * wy1),
    )
    for j, (xc, yc, wj) in enumerate(corners):
        valid = (xc >= 0.0) & (xc <= W - 1.0) & (yc >= 0.0) & (yc <= H - 1.0)
        xi = jnp.clip(xc, 0.0, W - 1.0).astype(jnp.int32)
        yi = jnp.clip(yc, 0.0, H - 1.0).astype(jnp.int32)
        idx_ref[j] = bbase + yi * W + xi
        w_ref[j] = jnp.where(valid, wj, 0.0)


_tc_index = pl.pallas_call(
    _tc_index_kernel,
    out_shape=(
        jax.ShapeDtypeStruct((4, G, KP), jnp.int32),
        jax.ShapeDtypeStruct((4, G, KP), jnp.float32),
    ),
)


def _make_sc_gather(nw):
    per_w = P // nw
    n_chunks = per_w // CH
    mesh = plsc.VectorSubcoreMesh(core_axis_name="c", subcore_axis_name="s")

    @functools.partial(
        pl.kernel,
        mesh=mesh,
        out_type=jax.ShapeDtypeStruct((P, C), jnp.float32),
        scratch_types=[
            pltpu.VMEM((n_chunks, 1, 4 * CH), jnp.int32),
            pltpu.VMEM((2, 4 * CH, LANES), jnp.float32),
            pltpu.VMEM((2, 4 * CH, C), jnp.float32),
            pltpu.VMEM((2, CH, C), jnp.float32),
            pltpu.SemaphoreType.DMA,
            pltpu.SemaphoreType.DMA,
            pltpu.SemaphoreType.DMA,
            pltpu.SemaphoreType.DMA,
            pltpu.SemaphoreType.DMA,
            pltpu.SemaphoreType.DMA,
        ],
    )
    def sc_fn(feats_hbm, idx_hbm, w_hbm, out_hbm, idx_all, w_v, rows_v, out_v,
              gs0, gs1, ws0, ws1, os0, os1):
        gsems = (gs0, gs1)
        wsems = (ws0, ws1)
        osems = (os0, os1)
        wid = lax.axis_index("s") * 2 + lax.axis_index("c")
        cbase = wid * n_chunks
        pbase = wid * per_w
        pltpu.sync_copy(idx_hbm.at[wid], idx_all)

        def start(ck, b):
            pltpu.async_copy(w_hbm.at[cbase + ck], w_v.at[b], wsems[b])
            pltpu.async_copy(feats_hbm.at[idx_all.at[ck, 0]], rows_v.at[b], gsems[b])

        def compute(ck, b):
            pltpu.make_async_copy(w_hbm.at[cbase + ck], w_v.at[b], wsems[b]).wait()
            pltpu.make_async_copy(
                feats_hbm.at[idx_all.at[ck, 0]], rows_v.at[b], gsems[b]).wait()

            @pl.when(ck >= 2)
            def _():
                pltpu.make_async_copy(
                    out_v.at[b], out_hbm.at[pl.ds(0, CH)], osems[b]).wait()

            def pbody(p, c2):
                wb0 = w_v[b, p, :]
                wb1 = w_v[b, CH + p, :]
                wb2 = w_v[b, 2 * CH + p, :]
                wb3 = w_v[b, 3 * CH + p, :]
                for cc in range(C // LANES):
                    s = pl.ds(cc * LANES, LANES)
                    acc = rows_v[b, p, s] * wb0
                    acc = acc + rows_v[b, CH + p, s] * wb1
                    acc = acc + rows_v[b, 2 * CH + p, s] * wb2
                    acc = acc + rows_v[b, 3 * CH + p, s] * wb3
                    out_v[b, p, s] = acc
                return c2

            lax.fori_loop(0, CH, pbody, 0)
            pltpu.async_copy(
                out_v.at[b], out_hbm.at[pl.ds(pbase + ck * CH, CH)], osems[b])

        start(0, 0)

        def pair(g, carry):
            ck = 2 * g
            start(ck + 1, 1)
            compute(ck, 0)
            start(ck + 2, 0)
            compute(ck + 1, 1)
            return carry

        lax.fori_loop(0, (n_chunks - 1) // 2, pair, 0)
        compute(n_chunks - 1, 0)
        pltpu.make_async_copy(out_v.at[0], out_hbm.at[pl.ds(0, CH)], osems[0]).wait()
        pltpu.make_async_copy(out_v.at[1], out_hbm.at[pl.ds(0, CH)], osems[1]).wait()

    return sc_fn


@functools.cache
def _sc_gather_cached():
    return _make_sc_gather(32)


def kernel(features, rois):
    feats_flat = jnp.transpose(features, (0, 2, 3, 1)).reshape(N * H * W, C)
    th = rois[:, 5] * SPATIAL_SCALE
    rt = jnp.stack(
        [
            rois[:, 0],
            rois[:, 1] * SPATIAL_SCALE,
            rois[:, 2] * SPATIAL_SCALE,
            rois[:, 3] * SPATIAL_SCALE,
            rois[:, 4] * SPATIAL_SCALE,
            jnp.cos(th),
            jnp.sin(th),
            jnp.zeros_like(th),
        ],
        axis=0,
    )
    rt = jnp.pad(rt, ((0, 0), (0, KP - K)))
    idx4, w4 = _tc_index(rt)
    n_chunks = P // CH // 32
    idx_sc = (
        idx4.reshape(4, P // CH, CH).transpose(1, 0, 2)
        .reshape(32, n_chunks, 1, 4 * CH)
    )
    w_sc = w4.reshape(4, P // CH, CH).transpose(1, 0, 2).reshape(P // CH, 4 * CH)
    w_sc = jnp.broadcast_to(w_sc[:, :, None], (P // CH, 4 * CH, LANES))
    out2 = _sc_gather_cached()(feats_flat, idx_sc, w_sc)
    out = out2.reshape(G, KP, C)[:, :K]
    return out.transpose(1, 2, 0).reshape(K, C, OUT_H, OUT_W)
